# baseline (device time: 53562 ns/iter reference)
import os

import numpy as np

import jax
import jax.numpy as jnp
from jax import lax
from jax.experimental import pallas as pl
from jax.experimental.pallas import tpu as pltpu

N_DEV = 16
B = 2
SQ = 256
D = 768
HQ = 4
DH = 64
HD = HQ * DH
ROWS = B * SQ
BH = B * HQ
QL = 8 * SQ
try:
    _PROBE = open(os.path.join(os.path.dirname(__file__), "PROBE")).read().strip()
except OSError:
    _PROBE = ""


def _rot_mat() -> np.ndarray:
    r = np.zeros((DH, DH), np.float32)
    for i in range(DH // 2):
        r[2 * i + 1, 2 * i] = -1.0
        r[2 * i, 2 * i + 1] = 1.0
    return np.kron(np.eye(HQ, dtype=np.float32), r)


def kernel(x, Wq, Wk, Wv, Wo):
    my_pos = lax.axis_index("i")
    x2d = x.reshape(ROWS, D)

    inv = 1.0 / (10000.0 ** (jnp.arange(0, DH, 2, dtype=jnp.float32) / DH))
    pos = (my_pos * SQ + jnp.arange(SQ, dtype=jnp.float32))[:, None] * inv[None, :]
    cos_big = jnp.tile(jnp.repeat(jnp.cos(pos), 2, axis=1), (B, HQ))
    sin_big = jnp.tile(jnp.repeat(jnp.sin(pos), 2, axis=1), (B, HQ))
    rot = jnp.asarray(_rot_mat(), dtype=jnp.bfloat16)

    def body(x_ref, wq_ref, wk_ref, wv_ref, wo_ref, cos_ref, sin_ref,
             rot_ref, out_ref,
             stage, qstage, cstage, kvbuf, qbuf, cbuf,
             kv_s, kv_r, q_s, q_r, c_s, c_r):
        if _PROBE == "null":
            out_ref[...] = jnp.zeros((ROWS, D), jnp.float32)
            return
        me = lax.axis_index("i")
        half = lax.div(me, 8)
        j_me = lax.rem(me, 8)

        xb = x_ref[...].astype(jnp.bfloat16)
        rot_m = rot_ref[...]

        def project(w_ref):
            return jax.lax.dot(xb, w_ref[...].astype(jnp.bfloat16),
                               preferred_element_type=jnp.float32)

        def rope(t):
            tr = jax.lax.dot(t.astype(jnp.bfloat16), rot_m,
                             preferred_element_type=jnp.float32)
            return (t * cos_ref[...] + tr * sin_ref[...]).astype(jnp.bfloat16)

        q2d = rope(project(wq_ref))
        k2d = rope(project(wk_ref))
        v2d = project(wv_ref).astype(jnp.bfloat16)

        for bh in range(BH):
            rs, cs = (bh // HQ) * SQ, (bh % HQ) * DH
            stage[bh, 0, :, :] = k2d[rs:rs + SQ, cs:cs + DH]
            stage[bh, 1, :, :] = v2d[rs:rs + SQ, cs:cs + DH]
            qstage[bh, :, :] = q2d[rs:rs + SQ, cs:cs + DH]

        barrier = pltpu.get_barrier_semaphore()
        for dev in range(N_DEV):
            pl.semaphore_signal(barrier, inc=1, device_id=(dev,),
                                device_id_type=pl.DeviceIdType.MESH)
        pl.semaphore_wait(barrier, N_DEV)

        sends = []

        def send(src_ref, dst_ref, ss, rs_, dev):
            d = pltpu.make_async_remote_copy(
                src_ref=src_ref, dst_ref=dst_ref, send_sem=ss, recv_sem=rs_,
                device_id=(dev,), device_id_type=pl.DeviceIdType.MESH)
            d.start()
            sends.append(d)

        def wait_in(dst_ref, rs_, dev):
            pltpu.make_async_remote_copy(
                src_ref=dst_ref, dst_ref=dst_ref, send_sem=rs_, recv_sem=rs_,
                device_id=(dev,),
                device_id_type=pl.DeviceIdType.MESH).wait_recv()

        for bh in range(BH):
            for ht in range(2):
                send(stage.at[bh], kvbuf.at[me],
                     kv_s.at[bh, ht], kv_r.at[me], ht * 8 + bh)
            send(qstage.at[bh], qbuf.at[j_me], q_s.at[bh], q_r.at[j_me],
                 half * 8 + bh)

        for j in range(8):
            wait_in(qbuf.at[j], q_r.at[j], half * 8 + j)
        qfull = jnp.concatenate([qbuf[j] for j in range(8)], axis=0)

        l = acc = None
        for s0 in range(0, N_DEV, 2):
            for src in (s0, s0 + 1):
                wait_in(kvbuf.at[src], kv_r.at[src], src)
            if _PROBE == "comm":
                continue
            k = jnp.concatenate([kvbuf[s0, 0], kvbuf[s0 + 1, 0]], axis=0)
            v = jnp.concatenate([kvbuf[s0, 1], kvbuf[s0 + 1, 1]], axis=0)
            s = jax.lax.dot_general(
                qfull, k, (((1,), (1,)), ((), ())),
                preferred_element_type=jnp.float32) * (0.125 * 1.4426950408889634)
            w = s if _PROBE == "noexp" else jnp.exp2(s)
            if l is None:
                l = jnp.sum(w, axis=1, keepdims=True)
                acc = jax.lax.dot(w.astype(jnp.bfloat16), v,
                                  preferred_element_type=jnp.float32)
            else:
                l = l + jnp.sum(w, axis=1, keepdims=True)
                acc = acc + jax.lax.dot(
                    w.astype(jnp.bfloat16), v,
                    preferred_element_type=jnp.float32)

        if _PROBE == "comm":
            ctx = qfull
        else:
            ctx = (acc / l).astype(jnp.bfloat16)

        for j in range(8):
            cstage[j, :, :] = ctx[j * SQ:(j + 1) * SQ, :]
            send(cstage.at[j], cbuf.at[j_me], c_s.at[j], c_r.at[j_me],
                 half * 8 + j)
        for bh in range(BH):
            wait_in(cbuf.at[bh], c_r.at[bh], half * 8 + bh)

        for d in sends:
            d.wait_send()

        ctx2d = jnp.concatenate(
            [jnp.concatenate([cbuf[b * HQ + hh] for hh in range(HQ)], axis=1)
             for b in range(B)], axis=0).astype(jnp.bfloat16)
        out_ref[...] = jax.lax.dot(ctx2d, wo_ref[...].astype(jnp.bfloat16),
                                   preferred_element_type=jnp.float32)

    out2d = pl.pallas_call(
        body,
        out_shape=jax.ShapeDtypeStruct((ROWS, D), jnp.float32),
        in_specs=[pl.BlockSpec(memory_space=pltpu.VMEM)] * 8,
        out_specs=pl.BlockSpec(memory_space=pltpu.VMEM),
        scratch_shapes=[
            pltpu.VMEM((BH, 2, SQ, DH), jnp.bfloat16),
            pltpu.VMEM((BH, SQ, DH), jnp.bfloat16),
            pltpu.VMEM((8, SQ, DH), jnp.bfloat16),
            pltpu.VMEM((N_DEV, 2, SQ, DH), jnp.bfloat16),
            pltpu.VMEM((8, SQ, DH), jnp.bfloat16),
            pltpu.VMEM((BH, SQ, DH), jnp.bfloat16),
            pltpu.SemaphoreType.DMA((BH, 2)),
            pltpu.SemaphoreType.DMA((N_DEV,)),
            pltpu.SemaphoreType.DMA((BH,)),
            pltpu.SemaphoreType.DMA((8,)),
            pltpu.SemaphoreType.DMA((8,)),
            pltpu.SemaphoreType.DMA((BH,)),
        ],
        compiler_params=(pltpu.CompilerParams() if _PROBE == "null"
                         else pltpu.CompilerParams(collective_id=0)),
    )(x2d, Wq, Wk, Wv, Wo, cos_big, sin_big, rot)
    return out2d.reshape(B, SQ, D)
